# baseline (speedup 1.0000x reference)
import functools

import jax
import jax.numpy as jnp
from jax import lax
from jax.experimental import pallas as pl
from jax.experimental.pallas import tpu as pltpu
from jax.experimental.pallas import tpu_sc as plsc

_NC, _NS, _L = 2, 16, 16


def _sc_body(col_hbm, row_hbm, out_hbm, tab_v, buf_v, sem):
    cid = lax.axis_index("c")
    sid = lax.axis_index("s")
    wid = sid * _NC + cid
    c0 = wid * 8

    @pl.when(wid < 16)
    def _():
        pltpu.sync_copy(col_hbm, tab_v)

    @pl.when(wid >= 16)
    def _():
        pltpu.sync_copy(row_hbm, tab_v)

    is_col = jnp.broadcast_to(wid < 16, (_L,))
    iota = lax.iota(jnp.int32, _L)
    ccol = (wid & 15) * 8

    for p in range(8):
        ccv = jnp.broadcast_to(ccol + p, (_L,)).astype(jnp.int32)
        for t in range(64):
            ridx = jnp.where(
                is_col,
                iota + (16 * (t & 1)),
                jnp.broadcast_to(jnp.int32(t >> 1), (_L,)),
            )
            buf_v[p, pl.ds(16 * t, _L)] = plsc.load_gather(
                tab_v, [ridx * 128 + ccv])

    copies = [
        pltpu.make_async_copy(buf_v, out_hbm.at[b, pl.ds(c0, 8)], sem)
        for b in range(4)
    ]
    for c in copies:
        c.start()
    for c in copies:
        c.wait()


@jax.jit
def _pos_embed(row_embed, col_embed):
    run = pl.kernel(
        _sc_body,
        out_type=jax.ShapeDtypeStruct((4, 256, 1024), jnp.float32),
        mesh=plsc.VectorSubcoreMesh(
            core_axis_name="c", subcore_axis_name="s",
            num_cores=_NC, num_subcores=_NS,
        ),
        scratch_types=[
            pltpu.VMEM((4096,), jnp.float32),
            pltpu.VMEM((8, 1024), jnp.float32),
            pltpu.SemaphoreType.DMA,
        ],
        compiler_params=pltpu.CompilerParams(needs_layout_passes=False),
    )
    out = run(col_embed[:32].reshape(4096), row_embed[:32].reshape(4096))
    return out.reshape(4, 256, 32, 32)


def kernel(x, row_embed, col_embed):
    del x
    return _pos_embed(row_embed, col_embed)


# speedup vs baseline: 1.2878x; 1.2878x over previous
import functools

import jax
import jax.numpy as jnp
from jax import lax
from jax.experimental import pallas as pl
from jax.experimental.pallas import tpu as pltpu
from jax.experimental.pallas import tpu_sc as plsc

_NC, _NS, _L = 2, 16, 16
_GDN = lax.GatherDimensionNumbers(
    offset_dims=(), collapsed_slice_dims=(0,), start_index_map=(0,))


def _sc_body(col_hbm, row_hbm, out_hbm, tab_v, buf_v, sem):
    cid = lax.axis_index("c")
    sid = lax.axis_index("s")
    wid = sid * _NC + cid
    c0 = wid * 8

    @pl.when(wid < 16)
    def _():
        pltpu.sync_copy(col_hbm, tab_v)

    @pl.when(wid >= 16)
    def _():
        pltpu.sync_copy(row_hbm, tab_v)

    iota = lax.iota(jnp.int32, _L)
    ccol = (wid & 15) * 8

    @pl.when(wid < 16)
    def _col_fill():
        for p in range(8):
            ccv = jnp.broadcast_to(ccol + p, (_L,)).astype(jnp.int32)
            v_lo = plsc.load_gather(tab_v, [iota * 128 + ccv])
            v_hi = plsc.load_gather(tab_v, [(iota + 16) * 128 + ccv])
            for t in range(64):
                buf_v[p, pl.ds(16 * t, _L)] = v_lo if (t & 1) == 0 else v_hi

    @pl.when(wid >= 16)
    def _row_fill():
        for p in range(8):
            ccv = jnp.broadcast_to(ccol + p, (_L,)).astype(jnp.int32)
            v_lo = plsc.load_gather(tab_v, [iota * 128 + ccv])
            v_hi = plsc.load_gather(tab_v, [(iota + 16) * 128 + ccv])
            for t in range(64):
                src = v_lo if (t >> 1) < 16 else v_hi
                idxv = jnp.full((_L, 1), (t >> 1) & 15, jnp.int32)
                buf_v[p, pl.ds(16 * t, _L)] = lax.gather(
                    src, idxv, _GDN, (1,),
                    mode=lax.GatherScatterMode.PROMISE_IN_BOUNDS)

    copies = [
        pltpu.make_async_copy(buf_v, out_hbm.at[b, pl.ds(c0, 8)], sem)
        for b in range(4)
    ]
    for c in copies:
        c.start()
    for c in copies:
        c.wait()


@jax.jit
def _pos_embed(row_embed, col_embed):
    run = pl.kernel(
        _sc_body,
        out_type=jax.ShapeDtypeStruct((4, 256, 1024), jnp.float32),
        mesh=plsc.VectorSubcoreMesh(
            core_axis_name="c", subcore_axis_name="s",
            num_cores=_NC, num_subcores=_NS,
        ),
        scratch_types=[
            pltpu.VMEM((4096,), jnp.float32),
            pltpu.VMEM((8, 1024), jnp.float32),
            pltpu.SemaphoreType.DMA,
        ],
        compiler_params=pltpu.CompilerParams(needs_layout_passes=False),
    )
    out = run(col_embed[:32].reshape(4096), row_embed[:32].reshape(4096))
    return out.reshape(4, 256, 32, 32)


def kernel(x, row_embed, col_embed):
    del x
    return _pos_embed(row_embed, col_embed)


# speedup vs baseline: 1.7215x; 1.3368x over previous
import jax
import jax.numpy as jnp
from jax.experimental import pallas as pl
from jax.experimental.pallas import tpu as pltpu


def _pos_body(col_ref, row_ref, out_ref, x3, y3, sem):
    col_t = col_ref[...].T
    row_t = row_ref[...].T
    x3[...] = jnp.broadcast_to(col_t[:, None, :], (128, 32, 32))
    xc = [pltpu.make_async_copy(x3, out_ref.at[b, 0:128], sem) for b in range(4)]
    for c in xc:
        c.start()
    y3[...] = jnp.broadcast_to(row_t[:, :, None], (128, 32, 32))
    yc = [pltpu.make_async_copy(y3, out_ref.at[b, 128:256], sem) for b in range(4)]
    for c in yc:
        c.start()
    for c in xc + yc:
        c.wait()


@jax.jit
def _pos_embed(row_embed, col_embed):
    return pl.pallas_call(
        _pos_body,
        in_specs=[
            pl.BlockSpec(memory_space=pltpu.VMEM),
            pl.BlockSpec(memory_space=pltpu.VMEM),
        ],
        out_specs=pl.BlockSpec(memory_space=pl.ANY),
        out_shape=jax.ShapeDtypeStruct((4, 256, 32, 32), jnp.float32),
        scratch_shapes=[
            pltpu.VMEM((128, 32, 32), jnp.float32),
            pltpu.VMEM((128, 32, 32), jnp.float32),
            pltpu.SemaphoreType.DMA,
        ],
    )(col_embed[:32], row_embed[:32])


def kernel(x, row_embed, col_embed):
    del x
    return _pos_embed(row_embed, col_embed)


# speedup vs baseline: 3.1129x; 1.8083x over previous
import jax
import jax.numpy as jnp
from jax.experimental import pallas as pl
from jax.experimental.pallas import tpu as pltpu


def _pos_body(col_ref, row_ref, out_ref, acc_ref):
    @pl.when(pl.program_id(0) == 0)
    def _():
        col_t = col_ref[...].T
        row_t = row_ref[...].T
        acc_ref[:128] = pltpu.repeat(col_t, 32, axis=1)
        for i in range(32):
            acc_ref[128:, pl.ds(32 * i, 32)] = jnp.broadcast_to(
                row_t[:, i : i + 1], (128, 32)
            )

    out_ref[0] = acc_ref[...]


@jax.jit
def _pos_embed(row_embed, col_embed):
    out = pl.pallas_call(
        _pos_body,
        grid=(4,),
        in_specs=[
            pl.BlockSpec((32, 128), lambda i: (0, 0)),
            pl.BlockSpec((32, 128), lambda i: (0, 0)),
        ],
        out_specs=pl.BlockSpec((1, 256, 1024), lambda i: (i, 0, 0)),
        out_shape=jax.ShapeDtypeStruct((4, 256, 1024), jnp.float32),
        scratch_shapes=[pltpu.VMEM((256, 1024), jnp.float32)],
    )(col_embed[:32], row_embed[:32])
    return out.reshape(4, 256, 32, 32)


def kernel(x, row_embed, col_embed):
    del x
    return _pos_embed(row_embed, col_embed)
